# baseline (speedup 1.0000x reference)
import functools

import jax
import jax.numpy as jnp
from jax import lax
from jax.experimental import pallas as pl
from jax.experimental.pallas import tpu as pltpu
from jax.experimental.pallas import tpu_sc as plsc

WORKER_NUM = 1000
TASK_NUM = 20000
ABILITY_NUM = 128
EDGE_TYPE = 2
COLS = TASK_NUM * EDGE_TYPE
TOTAL = WORKER_NUM * COLS

NC = 2
NS = 16
NW = NC * NS
BASE_ROWS = WORKER_NUM // NW
EXTRA = WORKER_NUM - BASE_ROWS * NW
PAD_ROWS = (BASE_ROWS + 1) * NW
LANES = 16
UNROLL = 20
INNER = COLS // (LANES * UNROLL)


def _params_block(wf_ref, w_ref, b_ref, lp2_ref, a_ref):
    x = jnp.dot(wf_ref[...], w_ref[...],
                preferred_element_type=jnp.float32) + b_ref[0, 0]
    p1 = jax.nn.sigmoid(x)
    p2 = 1.0 - p1
    lp1 = jnp.maximum(jnp.log(p1), -1e30)
    lp2 = jnp.maximum(jnp.log(p2), -1e30)
    lp2_ref[...] = jnp.broadcast_to(lp2, (WORKER_NUM, LANES))
    a_ref[...] = jnp.broadcast_to(lp1 - lp2, (WORKER_NUM, LANES))


def _sc_body(tau_hbm, lp2_hbm, a_hbm, out_hbm, tau_v, row_v, lp2_v, a_v):
    wid = lax.axis_index("s") * NC + lax.axis_index("c")
    base = BASE_ROWS * wid + jnp.minimum(wid, EXTRA)
    count = BASE_ROWS + (wid < EXTRA).astype(jnp.int32)

    pltpu.sync_copy(tau_hbm, tau_v)
    chunk = (BASE_ROWS + 1) * LANES
    pltpu.sync_copy(lp2_hbm.at[pl.ds(base * LANES, chunk)], lp2_v)
    pltpu.sync_copy(a_hbm.at[pl.ds(base * LANES, chunk)], a_v)

    def row_body(r, carry):
        lp2 = lp2_v[pl.ds(r * LANES, LANES)]
        a = a_v[pl.ds(r * LANES, LANES)]

        def col_body(k, carry2):
            off = k * (LANES * UNROLL)
            for u in range(UNROLL):
                t = tau_v[pl.ds(off + u * LANES, LANES)]
                row_v[pl.ds(off + u * LANES, LANES)] = jnp.exp(lp2 + t * a)
            return carry2

        lax.fori_loop(0, INNER, col_body, 0, unroll=False)
        pltpu.sync_copy(row_v, out_hbm.at[pl.ds((base + r) * COLS, COLS)])
        return carry

    lax.fori_loop(0, count, row_body, 0, unroll=False)


@jax.jit
def kernel(inputs, W, b, P0):
    wf = inputs[:WORKER_NUM]
    tau = inputs[WORKER_NUM:, :EDGE_TYPE].reshape(COLS)
    b2 = b.reshape(1, 1)

    lp2b, ab = pl.pallas_call(
        _params_block,
        in_specs=[
            pl.BlockSpec((WORKER_NUM, ABILITY_NUM), lambda: (0, 0)),
            pl.BlockSpec((ABILITY_NUM, 1), lambda: (0, 0)),
            pl.BlockSpec((1, 1), lambda: (0, 0)),
        ],
        out_specs=[
            pl.BlockSpec((WORKER_NUM, LANES), lambda: (0, 0)),
            pl.BlockSpec((WORKER_NUM, LANES), lambda: (0, 0)),
        ],
        out_shape=[
            jax.ShapeDtypeStruct((WORKER_NUM, LANES), jnp.float32),
            jax.ShapeDtypeStruct((WORKER_NUM, LANES), jnp.float32),
        ],
    )(wf, W, b2)

    pad = ((0, PAD_ROWS - WORKER_NUM), (0, 0))
    lp2_flat = jnp.pad(lp2b, pad).reshape(PAD_ROWS * LANES)
    a_flat = jnp.pad(ab, pad).reshape(PAD_ROWS * LANES)

    sc = functools.partial(
        pl.kernel,
        mesh=plsc.VectorSubcoreMesh(core_axis_name="c", subcore_axis_name="s"),
        out_type=jax.ShapeDtypeStruct((TOTAL,), jnp.float32),
        scratch_types=[
            pltpu.VMEM((COLS,), jnp.float32),
            pltpu.VMEM((COLS,), jnp.float32),
            pltpu.VMEM(((BASE_ROWS + 1) * LANES,), jnp.float32),
            pltpu.VMEM(((BASE_ROWS + 1) * LANES,), jnp.float32),
        ],
    )(_sc_body)
    out = sc(tau, lp2_flat, a_flat)
    return out.reshape(WORKER_NUM, TASK_NUM, EDGE_TYPE)


# speedup vs baseline: 15.6012x; 15.6012x over previous
import jax
import jax.numpy as jnp
from jax.experimental import pallas as pl
from jax.experimental.pallas import tpu as pltpu

WORKER_NUM = 1000
TASK_NUM = 20000
ABILITY_NUM = 128
EDGE_TYPE = 2
COLS = TASK_NUM * EDGE_TYPE

ROW_BLOCK = 40


def _decoder_block(wf_ref, w_ref, b_ref, tau_ref, out_ref):
    x = jnp.dot(wf_ref[...], w_ref[...],
                preferred_element_type=jnp.float32) + b_ref[0, 0]
    p1 = jax.nn.sigmoid(x)
    p2 = 1.0 - p1
    lp1 = jnp.maximum(jnp.log(p1), -1e30)
    lp2 = jnp.maximum(jnp.log(p2), -1e30)
    a = lp1 - lp2
    out_ref[...] = jnp.exp(lp2 + tau_ref[...] * a)


@jax.jit
def kernel(inputs, W, b, P0):
    wf = inputs[:WORKER_NUM]
    tau = inputs[WORKER_NUM:, :EDGE_TYPE].reshape(1, COLS)
    b2 = b.reshape(1, 1)
    grid = (WORKER_NUM // ROW_BLOCK,)
    out = pl.pallas_call(
        _decoder_block,
        grid=grid,
        in_specs=[
            pl.BlockSpec((ROW_BLOCK, ABILITY_NUM), lambda i: (i, 0)),
            pl.BlockSpec((ABILITY_NUM, 1), lambda i: (0, 0)),
            pl.BlockSpec((1, 1), lambda i: (0, 0)),
            pl.BlockSpec((1, COLS), lambda i: (0, 0)),
        ],
        out_specs=pl.BlockSpec((ROW_BLOCK, COLS), lambda i: (i, 0)),
        out_shape=jax.ShapeDtypeStruct((WORKER_NUM, COLS), jnp.float32),
        compiler_params=pltpu.CompilerParams(
            vmem_limit_bytes=100 * 1024 * 1024),
    )(wf, W, b2, tau)
    return out.reshape(WORKER_NUM, TASK_NUM, EDGE_TYPE)
